# baseline (speedup 1.0000x reference)
import functools

import jax
import jax.numpy as jnp
from jax import lax
from jax.experimental import pallas as pl
from jax.experimental.pallas import tpu as pltpu
from jax.experimental.pallas import tpu_sc as plsc

N = 10000
E = 320000
D = 128
L = 3

NC = 2
NS = 16
NW = NC * NS

K = 128
NCHUNK = 80
HC = NCHUNK // 2
EPW = K * NCHUNK
E_PAD = EPW * NW
NP = 10240
STRIPE = NP // NS

BLK = 1000
NB = N // BLK


def _agg_body(h_hbm, src_hbm, dst_hbm, zeros_hbm, out_hbm,
              src_v, dst_v, rows0, rows1, agg_sh,
              zsem, isem, gsem0, gsem1, ssem0, ssem1):
    c = lax.axis_index("c")
    s = lax.axis_index("s")
    wid = s * NC + c
    zero_dst = agg_sh.at[pl.ds(s * STRIPE, STRIPE)]
    pltpu.async_copy(zeros_hbm, zero_dst, zsem)

    def load_idx(half):
        pltpu.async_copy(src_hbm.at[wid].at[pl.ds(half * HC, HC)], src_v, isem)
        pltpu.async_copy(dst_hbm.at[wid].at[pl.ds(half * HC, HC)], dst_v, isem)

    def wait_idx(half):
        pltpu.make_async_copy(src_hbm.at[wid].at[pl.ds(half * HC, HC)],
                              src_v, isem).wait()
        pltpu.make_async_copy(dst_hbm.at[wid].at[pl.ds(half * HC, HC)],
                              dst_v, isem).wait()

    load_idx(0)
    wait_idx(0)
    pltpu.make_async_copy(zeros_hbm, zero_dst, zsem).wait()
    plsc.subcore_barrier()

    bufs = (rows0, rows1)
    gsems = (gsem0, gsem1)
    ssems = (ssem0, ssem1)

    def start_gather(b, j):
        pltpu.async_copy(h_hbm.at[src_v.at[j]], bufs[b], gsems[b])

    def wait_gather(b, j):
        pltpu.make_async_copy(h_hbm.at[src_v.at[j]], bufs[b], gsems[b]).wait()

    def start_scatter(b, j):
        pltpu.async_copy(bufs[b], agg_sh.at[dst_v.at[j]], ssems[b], add=True)

    def wait_scatter(b, j):
        pltpu.make_async_copy(bufs[b], agg_sh.at[dst_v.at[j]],
                              ssems[b]).wait()

    def run_half():
        start_gather(0, 0)
        wait_gather(0, 0)
        start_scatter(0, 0)
        start_gather(1, 1)

        def body(j, carry):
            def step(cur, oth):
                wait_gather(cur, j)
                start_scatter(cur, j)
                wait_scatter(oth, j - 1)
                start_gather(oth, j + 1)

            @pl.when(j % 2 == 0)
            def _():
                step(0, 1)

            @pl.when(j % 2 == 1)
            def _():
                step(1, 0)

            return carry

        lax.fori_loop(1, HC - 1, body, 0)
        wait_gather(1, HC - 1)
        start_scatter(1, HC - 1)
        wait_scatter(0, HC - 2)
        wait_scatter(1, HC - 1)

    run_half()
    load_idx(1)
    wait_idx(1)
    run_half()
    plsc.subcore_barrier()
    pltpu.sync_copy(agg_sh.at[pl.ds(s * STRIPE, STRIPE)],
                    out_hbm.at[pl.ds(c * NP + s * STRIPE, STRIPE)])


@functools.cache
def _agg_kernel():
    return pl.kernel(
        _agg_body,
        out_type=jax.ShapeDtypeStruct((NC * NP, D), jnp.float32),
        mesh=plsc.VectorSubcoreMesh(core_axis_name="c", subcore_axis_name="s",
                                    num_cores=NC, num_subcores=NS),
        scratch_types=[
            pltpu.VMEM((HC, K), jnp.int32),
            pltpu.VMEM((HC, K), jnp.int32),
            pltpu.VMEM((K, D), jnp.float32),
            pltpu.VMEM((K, D), jnp.float32),
            pltpu.VMEM_SHARED((NP, D), jnp.float32),
            pltpu.SemaphoreType.DMA,
            pltpu.SemaphoreType.DMA,
            pltpu.SemaphoreType.DMA,
            pltpu.SemaphoreType.DMA,
            pltpu.SemaphoreType.DMA,
            pltpu.SemaphoreType.DMA,
        ],
    )


def _agg(h, src_p, dst_p, zeros_stripe):
    return _agg_kernel()(h, src_p, dst_p, zeros_stripe)


def _lin_body(x_ref, w_ref, b_ref, o_ref):
    o_ref[...] = (
        jnp.dot(x_ref[...], w_ref[...], preferred_element_type=jnp.float32)
        + b_ref[...]
    )


def _linear(x, w, b):
    return pl.pallas_call(
        _lin_body,
        grid=(NB,),
        in_specs=[
            pl.BlockSpec((BLK, D), lambda i: (i, 0)),
            pl.BlockSpec((D, D), lambda i: (0, 0)),
            pl.BlockSpec((1, D), lambda i: (0, 0)),
        ],
        out_specs=pl.BlockSpec((BLK, D), lambda i: (i, 0)),
        out_shape=jax.ShapeDtypeStruct((N, D), jnp.float32),
    )(x, w, b)


def _mlp_body(h_ref, agg_ref, w1_ref, b1_ref, w2_ref, b2_ref,
              z_ref, stats_ref, acc_ref):
    i = pl.program_id(0)
    z = h_ref[...] + agg_ref[0] + agg_ref[1]
    z = jnp.maximum(
        jnp.dot(z, w1_ref[...], preferred_element_type=jnp.float32) + b1_ref[...], 0.0)
    z = jnp.maximum(
        jnp.dot(z, w2_ref[...], preferred_element_type=jnp.float32) + b2_ref[...],
        0.0)
    z_ref[...] = z

    @pl.when(i == 0)
    def _():
        acc_ref[...] = jnp.zeros_like(acc_ref)

    acc_ref[0:1] += jnp.sum(z, axis=0, keepdims=True)
    acc_ref[1:2] += jnp.sum(z * z, axis=0, keepdims=True)
    stats_ref[...] = acc_ref[...]


def _mlp(h, parts, w1, b1, w2, b2):
    return pl.pallas_call(
        _mlp_body,
        grid=(NB,),
        in_specs=[
            pl.BlockSpec((BLK, D), lambda i: (i, 0)),
            pl.BlockSpec((NC, BLK, D), lambda i: (0, i, 0)),
            pl.BlockSpec((D, D), lambda i: (0, 0)),
            pl.BlockSpec((1, D), lambda i: (0, 0)),
            pl.BlockSpec((D, D), lambda i: (0, 0)),
            pl.BlockSpec((1, D), lambda i: (0, 0)),
        ],
        out_specs=[
            pl.BlockSpec((BLK, D), lambda i: (i, 0)),
            pl.BlockSpec((2, D), lambda i: (0, 0)),
        ],
        out_shape=[
            jax.ShapeDtypeStruct((N, D), jnp.float32),
            jax.ShapeDtypeStruct((2, D), jnp.float32),
        ],
        scratch_shapes=[pltpu.VMEM((2, D), jnp.float32)],
    )(h, parts, w1, b1, w2, b2)


def _bn_body(z_ref, stats_ref, g_ref, be_ref, o_ref):
    inv_n = jnp.float32(1.0 / N)
    mean = stats_ref[0:1] * inv_n
    var = stats_ref[1:2] * inv_n - mean * mean
    scale = g_ref[...] * lax.rsqrt(var + 1e-5)
    o_ref[...] = jnp.tanh((z_ref[...] - mean) * scale + be_ref[...])


def _bn(z, stats, g, be):
    return pl.pallas_call(
        _bn_body,
        grid=(NB,),
        in_specs=[
            pl.BlockSpec((BLK, D), lambda i: (i, 0)),
            pl.BlockSpec((2, D), lambda i: (0, 0)),
            pl.BlockSpec((1, D), lambda i: (0, 0)),
            pl.BlockSpec((1, D), lambda i: (0, 0)),
        ],
        out_specs=pl.BlockSpec((BLK, D), lambda i: (i, 0)),
        out_shape=jax.ShapeDtypeStruct((N, D), jnp.float32),
    )(z, stats, g, be)


def kernel(x, edge_index, W0, b0, W1, B1, W2, B2, G, Be):
    src = edge_index[0]
    dst = edge_index[1]
    pad = E_PAD - E
    src_p = jnp.concatenate([src, jnp.zeros((pad,), jnp.int32)])
    src_p = src_p.reshape(NW, NCHUNK, K)
    dst_p = jnp.concatenate([dst, jnp.full((pad,), N, jnp.int32)])
    dst_p = dst_p.reshape(NW, NCHUNK, K)
    zeros_stripe = jnp.zeros((STRIPE, D), jnp.float32)

    h = _linear(x, W0, b0.reshape(1, D))
    outs = [x]
    for l in range(L):
        parts = _agg(h, src_p, dst_p, zeros_stripe).reshape(NC, NP, D)
        z, stats = _mlp(h, parts, W1[l], B1[l].reshape(1, D),
                        W2[l], B2[l].reshape(1, D))
        h = _bn(z, stats, G[l].reshape(1, D), Be[l].reshape(1, D))
        outs.append(h)
    return tuple(outs)


# speedup vs baseline: 1.1368x; 1.1368x over previous
import functools

import jax
import jax.numpy as jnp
from jax import lax
from jax.experimental import pallas as pl
from jax.experimental.pallas import tpu as pltpu
from jax.experimental.pallas import tpu_sc as plsc

N = 10000
E = 320000
D = 128
L = 3

NC = 2
NS = 16
NW = NC * NS

K = 128
FAST_CORE = 0
CF = 128
CS = 32
HF = CF // 2
HS = CS
E_PAD = NS * (CF + CS) * K
NP = 10112
STRIPE = NP // NS

BLK = 1000
NB = N // BLK


def _agg_body(h_hbm, srcf_hbm, dstf_hbm, srcs_hbm, dsts_hbm, zeros_hbm,
              out_hbm, src_v, dst_v, rows0, rows1, agg_sh,
              zsem, isem, gsem0, gsem1, ssem0, ssem1):
    c = lax.axis_index("c")
    s = lax.axis_index("s")
    zero_dst = agg_sh.at[pl.ds(s * STRIPE, STRIPE)]
    pltpu.async_copy(zeros_hbm, zero_dst, zsem)

    bufs = (rows0, rows1)
    gsems = (gsem0, gsem1)
    ssems = (ssem0, ssem1)

    def start_gather(b, j):
        pltpu.async_copy(h_hbm.at[src_v.at[j]], bufs[b], gsems[b])

    def wait_gather(b, j):
        pltpu.make_async_copy(h_hbm.at[src_v.at[j]], bufs[b], gsems[b]).wait()

    def start_scatter(b, j):
        pltpu.async_copy(bufs[b], agg_sh.at[dst_v.at[j]], ssems[b], add=True)

    def wait_scatter(b, j):
        pltpu.make_async_copy(bufs[b], agg_sh.at[dst_v.at[j]],
                              ssems[b]).wait()

    def run_half(hc):
        start_gather(0, 0)
        wait_gather(0, 0)
        start_scatter(0, 0)
        start_gather(1, 1)

        def body(j, carry):
            def step(cur, oth):
                wait_gather(cur, j)
                start_scatter(cur, j)
                wait_scatter(oth, j - 1)
                start_gather(oth, j + 1)

            @pl.when(j % 2 == 0)
            def _():
                step(0, 1)

            @pl.when(j % 2 == 1)
            def _():
                step(1, 0)

            return carry

        lax.fori_loop(1, hc - 1, body, 0)
        wait_gather(1, hc - 1)
        start_scatter(1, hc - 1)
        wait_scatter(0, hc - 2)
        wait_scatter(1, hc - 1)

    def run_core(src_hbm, dst_hbm, hc, nloads):
        def idx_copies(half):
            src_d = src_v if hc == HF else src_v.at[pl.ds(0, hc)]
            dst_d = dst_v if hc == HF else dst_v.at[pl.ds(0, hc)]
            return (
                pltpu.make_async_copy(
                    src_hbm.at[s].at[pl.ds(half * hc, hc)], src_d, isem),
                pltpu.make_async_copy(
                    dst_hbm.at[s].at[pl.ds(half * hc, hc)], dst_d, isem),
            )

        def load_idx(half):
            for cp in idx_copies(half):
                cp.start()

        def wait_idx(half):
            for cp in idx_copies(half):
                cp.wait()

        load_idx(0)
        wait_idx(0)
        pltpu.make_async_copy(zeros_hbm, zero_dst, zsem).wait()
        plsc.subcore_barrier()
        run_half(hc)
        for half in range(1, nloads):
            load_idx(half)
            wait_idx(half)
            run_half(hc)

    @pl.when(c == FAST_CORE)
    def _():
        run_core(srcf_hbm, dstf_hbm, HF, 2)

    @pl.when(c != FAST_CORE)
    def _():
        run_core(srcs_hbm, dsts_hbm, HS, 1)

    plsc.subcore_barrier()
    pltpu.sync_copy(agg_sh.at[pl.ds(s * STRIPE, STRIPE)],
                    out_hbm.at[pl.ds(c * NP + s * STRIPE, STRIPE)])


@functools.cache
def _agg_kernel():
    return pl.kernel(
        _agg_body,
        out_type=jax.ShapeDtypeStruct((NC * NP, D), jnp.float32),
        mesh=plsc.VectorSubcoreMesh(core_axis_name="c", subcore_axis_name="s",
                                    num_cores=NC, num_subcores=NS),
        scratch_types=[
            pltpu.VMEM((HF, K), jnp.int32),
            pltpu.VMEM((HF, K), jnp.int32),
            pltpu.VMEM((K, D), jnp.float32),
            pltpu.VMEM((K, D), jnp.float32),
            pltpu.VMEM_SHARED((NP, D), jnp.float32),
            pltpu.SemaphoreType.DMA,
            pltpu.SemaphoreType.DMA,
            pltpu.SemaphoreType.DMA,
            pltpu.SemaphoreType.DMA,
            pltpu.SemaphoreType.DMA,
            pltpu.SemaphoreType.DMA,
        ],
    )


def _agg(h, src_f, dst_f, src_s, dst_s, zeros_stripe):
    return _agg_kernel()(h, src_f, dst_f, src_s, dst_s, zeros_stripe)


def _lin_body(x_ref, w_ref, b_ref, o_ref):
    o_ref[...] = (
        jnp.dot(x_ref[...], w_ref[...], preferred_element_type=jnp.float32)
        + b_ref[...]
    )


def _linear(x, w, b):
    return pl.pallas_call(
        _lin_body,
        grid=(NB,),
        in_specs=[
            pl.BlockSpec((BLK, D), lambda i: (i, 0)),
            pl.BlockSpec((D, D), lambda i: (0, 0)),
            pl.BlockSpec((1, D), lambda i: (0, 0)),
        ],
        out_specs=pl.BlockSpec((BLK, D), lambda i: (i, 0)),
        out_shape=jax.ShapeDtypeStruct((N, D), jnp.float32),
    )(x, w, b)


def _mlp_body(h_ref, agg_ref, w1_ref, b1_ref, w2_ref, b2_ref,
              z_ref, stats_ref, acc_ref):
    i = pl.program_id(0)
    z = h_ref[...] + agg_ref[0] + agg_ref[1]
    z = jnp.maximum(
        jnp.dot(z, w1_ref[...], preferred_element_type=jnp.float32) + b1_ref[...], 0.0)
    z = jnp.maximum(
        jnp.dot(z, w2_ref[...], preferred_element_type=jnp.float32) + b2_ref[...],
        0.0)
    z_ref[...] = z

    @pl.when(i == 0)
    def _():
        acc_ref[...] = jnp.zeros_like(acc_ref)

    acc_ref[0:1] += jnp.sum(z, axis=0, keepdims=True)
    acc_ref[1:2] += jnp.sum(z * z, axis=0, keepdims=True)
    stats_ref[...] = acc_ref[...]


def _mlp(h, parts, w1, b1, w2, b2):
    return pl.pallas_call(
        _mlp_body,
        grid=(NB,),
        in_specs=[
            pl.BlockSpec((BLK, D), lambda i: (i, 0)),
            pl.BlockSpec((NC, BLK, D), lambda i: (0, i, 0)),
            pl.BlockSpec((D, D), lambda i: (0, 0)),
            pl.BlockSpec((1, D), lambda i: (0, 0)),
            pl.BlockSpec((D, D), lambda i: (0, 0)),
            pl.BlockSpec((1, D), lambda i: (0, 0)),
        ],
        out_specs=[
            pl.BlockSpec((BLK, D), lambda i: (i, 0)),
            pl.BlockSpec((2, D), lambda i: (0, 0)),
        ],
        out_shape=[
            jax.ShapeDtypeStruct((N, D), jnp.float32),
            jax.ShapeDtypeStruct((2, D), jnp.float32),
        ],
        scratch_shapes=[pltpu.VMEM((2, D), jnp.float32)],
    )(h, parts, w1, b1, w2, b2)


def _bn_body(z_ref, stats_ref, g_ref, be_ref, o_ref):
    inv_n = jnp.float32(1.0 / N)
    mean = stats_ref[0:1] * inv_n
    var = stats_ref[1:2] * inv_n - mean * mean
    scale = g_ref[...] * lax.rsqrt(var + 1e-5)
    o_ref[...] = jnp.tanh((z_ref[...] - mean) * scale + be_ref[...])


def _bn(z, stats, g, be):
    return pl.pallas_call(
        _bn_body,
        grid=(NB,),
        in_specs=[
            pl.BlockSpec((BLK, D), lambda i: (i, 0)),
            pl.BlockSpec((2, D), lambda i: (0, 0)),
            pl.BlockSpec((1, D), lambda i: (0, 0)),
            pl.BlockSpec((1, D), lambda i: (0, 0)),
        ],
        out_specs=pl.BlockSpec((BLK, D), lambda i: (i, 0)),
        out_shape=jax.ShapeDtypeStruct((N, D), jnp.float32),
    )(z, stats, g, be)


def kernel(x, edge_index, W0, b0, W1, B1, W2, B2, G, Be):
    src = edge_index[0]
    dst = edge_index[1]
    pad = E_PAD - E
    nf = NS * CF * K
    src_p = jnp.concatenate([src, jnp.zeros((pad,), jnp.int32)])
    dst_p = jnp.concatenate([dst, jnp.full((pad,), N, jnp.int32)])
    src_f = src_p[:nf].reshape(NS, CF, K)
    dst_f = dst_p[:nf].reshape(NS, CF, K)
    src_s = src_p[nf:].reshape(NS, CS, K)
    dst_s = dst_p[nf:].reshape(NS, CS, K)
    zeros_stripe = jnp.zeros((STRIPE, D), jnp.float32)

    h = _linear(x, W0, b0.reshape(1, D))
    outs = [x]
    for l in range(L):
        parts = _agg(h, src_f, dst_f, src_s, dst_s,
                     zeros_stripe).reshape(NC, NP, D)
        z, stats = _mlp(h, parts, W1[l], B1[l].reshape(1, D),
                        W2[l], B2[l].reshape(1, D))
        h = _bn(z, stats, G[l].reshape(1, D), Be[l].reshape(1, D))
        outs.append(h)
    return tuple(outs)
